# baseline (speedup 1.0000x reference)
import functools

import jax
import jax.numpy as jnp
from jax import lax
from jax.experimental import pallas as pl
from jax.experimental.pallas import tpu as pltpu
from jax.experimental.pallas import tpu_sc as plsc

HIDDEN = 128
BATCH = 16384

_info = plsc.get_sparse_core_info()
_NC, _NS = _info.num_cores, _info.num_subcores
_NW = _NC * _NS
_BPW = BATCH // _NW
_CHUNK = 128
_NCHUNK = _BPW // _CHUNK
_NBUF = 2

_mesh = plsc.VectorSubcoreMesh(core_axis_name="c", subcore_axis_name="s")


@functools.partial(
    pl.kernel,
    mesh=_mesh,
    out_type=jax.ShapeDtypeStruct((BATCH // _CHUNK, _CHUNK, HIDDEN), jnp.float32),
    scratch_types=[
        pltpu.VMEM((_NCHUNK, _CHUNK), jnp.int32),
        pltpu.VMEM_SHARED((4, HIDDEN), jnp.float32),
        pltpu.VMEM((_NBUF, _CHUNK, HIDDEN), jnp.float32),
        pltpu.SemaphoreType.DMA,
        pltpu.SemaphoreType.DMA,
        pltpu.SemaphoreType.DMA,
    ],
)
def _emb_lookup(idx_hbm, table_hbm, out_hbm, idx_v, table_sh, rows_v, gsem, ssem, psem):
    wid = lax.axis_index("s") * _NC + lax.axis_index("c")
    base = wid * _NCHUNK
    pltpu.async_copy(table_hbm, table_sh, psem)
    pltpu.async_copy(idx_hbm.at[pl.ds(base, _NCHUNK)], idx_v, psem)
    pltpu.make_async_copy(table_hbm, table_sh, psem).wait()
    pltpu.make_async_copy(idx_hbm.at[pl.ds(base, _NCHUNK)], idx_v, psem).wait()
    for j in range(_NBUF):
        pltpu.async_copy(table_sh.at[idx_v.at[j]], rows_v.at[j], gsem)
    for j in range(_NCHUNK):
        b = j % _NBUF
        pltpu.make_async_copy(table_sh.at[idx_v.at[j]], rows_v.at[b], gsem).wait()
        pltpu.async_copy(rows_v.at[b], out_hbm.at[base + j], ssem)
        if j + _NBUF < _NCHUNK:
            pltpu.make_async_copy(rows_v.at[b], out_hbm.at[base + j], ssem).wait()
            pltpu.async_copy(table_sh.at[idx_v.at[j + _NBUF]], rows_v.at[b], gsem)
    for j in range(_NCHUNK - _NBUF, _NCHUNK):
        pltpu.make_async_copy(rows_v.at[j % _NBUF], out_hbm.at[base + j], ssem).wait()


def kernel(session_types, session_emb_weight):
    idx = session_types.astype(jnp.int32).reshape(BATCH // _CHUNK, _CHUNK)
    out = _emb_lookup(idx, session_emb_weight)
    return out.reshape(BATCH, HIDDEN)
